# baseline (speedup 1.0000x reference)
import functools

import jax
import jax.numpy as jnp
from jax import lax
from jax.experimental import pallas as pl
from jax.experimental.pallas import tpu as pltpu
from jax.experimental.pallas import tpu_sc as plsc

DIM = 64
WIDE = 128
LANES = 16
NUM_CORES = 2
NUM_SUBCORES = 16
NUM_WORKERS = NUM_CORES * NUM_SUBCORES
CHUNK = 400


@functools.partial(jax.jit, static_argnames=("total",))
def _gather_rows(ids_flat, table_wide, total):
    per_w = total // NUM_WORKERS
    n_chunks = per_w // CHUNK
    mesh = plsc.VectorSubcoreMesh(core_axis_name="c", subcore_axis_name="s")

    @functools.partial(
        pl.kernel,
        out_type=jax.ShapeDtypeStruct((total, DIM), jnp.float32),
        mesh=mesh,
        scratch_types=[
            pltpu.VMEM((CHUNK,), jnp.int32),
            pltpu.VMEM((CHUNK, WIDE), jnp.float32),
            pltpu.VMEM((CHUNK, DIM), jnp.float32),
            pltpu.SemaphoreType.DMA,
        ],
    )
    def body(ids_hbm, table_hbm, out_hbm, idx_v, rows_v, rows64_v, sem):
        wid = lax.axis_index("s") * NUM_CORES + lax.axis_index("c")
        base = wid * per_w

        @pl.loop(0, n_chunks)
        def _chunk(i):
            off = base + i * CHUNK
            pltpu.sync_copy(ids_hbm.at[pl.ds(off, CHUNK)], idx_v)
            pltpu.async_copy(table_hbm.at[idx_v], rows_v, sem).wait()

            @pl.loop(0, CHUNK, unroll=8)
            def _row(j):
                for k in range(DIM // LANES):
                    rows64_v[j, pl.ds(k * LANES, LANES)] = (
                        rows_v[j, pl.ds(k * LANES, LANES)])

            pltpu.sync_copy(rows64_v, out_hbm.at[pl.ds(off, CHUNK)])

    return body(ids_flat, table_wide)


def kernel(ids, pads, table):
    B, L = ids.shape
    total = B * L
    table_wide = jnp.pad(table, ((0, 0), (0, WIDE - DIM)))
    rows = _gather_rows(ids.reshape(total), table_wide, total)
    return rows.reshape(B, L, DIM), pads


# speedup vs baseline: 1.4022x; 1.4022x over previous
import functools

import jax
import jax.numpy as jnp
from jax import lax
from jax.experimental import pallas as pl
from jax.experimental.pallas import tpu as pltpu
from jax.experimental.pallas import tpu_sc as plsc

DIM = 64
WIDE = 128
NUM_CORES = 2
NUM_SUBCORES = 16
NUM_WORKERS = NUM_CORES * NUM_SUBCORES
CHUNK = 400


@functools.partial(jax.jit, static_argnames=("total",))
def _gather_rows(ids_flat, table_wide, total):
    per_w = total // NUM_WORKERS
    n_chunks = per_w // CHUNK
    n_pairs = n_chunks // 2
    mesh = plsc.VectorSubcoreMesh(core_axis_name="c", subcore_axis_name="s")

    @functools.partial(
        pl.kernel,
        out_type=jax.ShapeDtypeStruct((total, WIDE), jnp.float32),
        mesh=mesh,
        scratch_types=[
            pltpu.VMEM((CHUNK,), jnp.int32),
            pltpu.VMEM((CHUNK,), jnp.int32),
            pltpu.VMEM((CHUNK, WIDE), jnp.float32),
            pltpu.VMEM((CHUNK, WIDE), jnp.float32),
            pltpu.SemaphoreType.DMA,
            pltpu.SemaphoreType.DMA,
        ],
    )
    def body(ids_hbm, table_hbm, out_hbm, idx0_v, idx1_v, rows0_v, rows1_v,
             sem0, sem1):
        wid = lax.axis_index("s") * NUM_CORES + lax.axis_index("c")
        base = wid * per_w

        pltpu.sync_copy(ids_hbm.at[pl.ds(base, CHUNK)], idx0_v)
        pltpu.async_copy(table_hbm.at[idx0_v], rows0_v, sem0)

        @pl.loop(0, n_pairs)
        def _pair(j):
            i0 = 2 * j
            off0 = base + i0 * CHUNK
            off1 = off0 + CHUNK

            pltpu.sync_copy(ids_hbm.at[pl.ds(off1, CHUNK)], idx1_v)
            pltpu.async_copy(table_hbm.at[idx1_v], rows1_v, sem1)

            pltpu.make_async_copy(table_hbm.at[idx0_v], rows0_v, sem0).wait()
            pltpu.sync_copy(rows0_v, out_hbm.at[pl.ds(off0, CHUNK)])

            @pl.when(j < n_pairs - 1)
            def _():
                off2 = off1 + CHUNK
                pltpu.sync_copy(ids_hbm.at[pl.ds(off2, CHUNK)], idx0_v)
                pltpu.async_copy(table_hbm.at[idx0_v], rows0_v, sem0)

            pltpu.make_async_copy(table_hbm.at[idx1_v], rows1_v, sem1).wait()
            pltpu.sync_copy(rows1_v, out_hbm.at[pl.ds(off1, CHUNK)])

    return body(ids_flat, table_wide)


def kernel(ids, pads, table):
    B, L = ids.shape
    total = B * L
    table_wide = jnp.pad(table, ((0, 0), (0, WIDE - DIM)))
    rows = _gather_rows(ids.reshape(total), table_wide, total)
    return rows[:, :DIM].reshape(B, L, DIM), pads


# speedup vs baseline: 1.4063x; 1.0029x over previous
import functools

import jax
import jax.numpy as jnp
from jax import lax
from jax.experimental import pallas as pl
from jax.experimental.pallas import tpu as pltpu
from jax.experimental.pallas import tpu_sc as plsc

DIM = 64
WIDE = 128
NUM_CORES = 2
NUM_SUBCORES = 16
NUM_WORKERS = NUM_CORES * NUM_SUBCORES
CHUNK = 400


@functools.partial(jax.jit, static_argnames=("total",))
def _gather_rows(ids_flat, table_wide, total):
    per_w = total // NUM_WORKERS
    n_chunks = per_w // CHUNK
    n_pairs = n_chunks // 2
    mesh = plsc.VectorSubcoreMesh(core_axis_name="c", subcore_axis_name="s")

    @functools.partial(
        pl.kernel,
        out_type=jax.ShapeDtypeStruct((total, WIDE), jnp.float32),
        mesh=mesh,
        scratch_types=[
            pltpu.VMEM((per_w,), jnp.int32),
            pltpu.VMEM((CHUNK, WIDE), jnp.float32),
            pltpu.VMEM((CHUNK, WIDE), jnp.float32),
            pltpu.SemaphoreType.DMA,
            pltpu.SemaphoreType.DMA,
        ],
    )
    def body(ids_hbm, table_hbm, out_hbm, idx_v, rows0_v, rows1_v, sem0, sem1):
        wid = lax.axis_index("s") * NUM_CORES + lax.axis_index("c")
        base = wid * per_w

        pltpu.sync_copy(ids_hbm.at[pl.ds(base, per_w)], idx_v)

        pltpu.async_copy(
            table_hbm.at[idx_v.at[pl.ds(0, CHUNK)]], rows0_v, sem0)

        @pl.loop(0, n_pairs)
        def _pair(j):
            i0 = 2 * j
            off0 = base + i0 * CHUNK
            off1 = off0 + CHUNK

            pltpu.async_copy(
                table_hbm.at[idx_v.at[pl.ds((i0 + 1) * CHUNK, CHUNK)]],
                rows1_v, sem1)

            pltpu.make_async_copy(
                table_hbm.at[idx_v.at[pl.ds(0, CHUNK)]], rows0_v, sem0).wait()
            pltpu.sync_copy(rows0_v, out_hbm.at[pl.ds(off0, CHUNK)])

            @pl.when(j < n_pairs - 1)
            def _():
                pltpu.async_copy(
                    table_hbm.at[idx_v.at[pl.ds((i0 + 2) * CHUNK, CHUNK)]],
                    rows0_v, sem0)

            pltpu.make_async_copy(
                table_hbm.at[idx_v.at[pl.ds(0, CHUNK)]], rows1_v, sem1).wait()
            pltpu.sync_copy(rows1_v, out_hbm.at[pl.ds(off1, CHUNK)])

    return body(ids_flat, table_wide)


def kernel(ids, pads, table):
    B, L = ids.shape
    total = B * L
    table_wide = jnp.pad(table, ((0, 0), (0, WIDE - DIM)))
    rows = _gather_rows(ids.reshape(total), table_wide, total)
    return rows[:, :DIM].reshape(B, L, DIM), pads
